# baseline (speedup 1.0000x reference)
import jax
import jax.numpy as jnp
from jax.experimental import pallas as pl
from jax.experimental.pallas import tpu as pltpu

_NEG = -1e30


def _lrelu(x):
    return jnp.where(x >= 0, x, 0.2 * x)


def _softmax_rows(score, adj_bool):
    masked = jnp.where(adj_bool, score, _NEG)
    m = jnp.max(masked, axis=1, keepdims=True)
    e = jnp.exp(masked - m)
    return e / jnp.sum(e, axis=1, keepdims=True)


def _egat_kernel(pni_ref, pei_ref,
                 H_ref, EC_ref, EL_ref, AH_ref, AE_ref, MEr_ref, MEany_ref,
                 maskF_ref,
                 WH_ref, WHb_ref, WEC_ref, WECb_ref, WEL_ref, WELb_ref,
                 aC_ref, aL_ref, bC_ref, bL_ref,
                 Hn_ref, ECn_ref, ELn_ref, Hm_ref,
                 attnC_sc, attnL_sc, rme_sc, dma_sem):
    N = 64
    M = 50
    D = 256
    b = pl.program_id(0)

    copies = []
    for k in range(8):
        i = pni_ref[k]
        c = pltpu.make_async_copy(
            MEany_ref.at[b, pl.ds(i * N, N), :],
            rme_sc.at[pl.ds(k * N, N), :],
            dma_sem,
        )
        c.start()
        copies.append(c)

    H = H_ref[0]
    EC = EC_ref[0]
    EL = EL_ref[0]
    MEr = MEr_ref[0]
    aC = aC_ref[...]
    aL = aL_ref[...]
    bC = bC_ref[...]
    bL = bL_ref[...]
    WECb = WECb_ref[...].reshape(1, D)
    WELb = WELb_ref[...].reshape(1, D)

    Ht = jnp.dot(H, WH_ref[...], preferred_element_type=jnp.float32) + WHb_ref[...].reshape(1, D)
    FC = jnp.dot(EC, WEC_ref[...], preferred_element_type=jnp.float32)
    FL = jnp.dot(EL, WEL_ref[...], preferred_element_type=jnp.float32)

    wvC = jnp.dot(WEC_ref[...], aC[2 * D:, :], preferred_element_type=jnp.float32)
    wvL = jnp.dot(WEL_ref[...], aL[2 * D:, :], preferred_element_type=jnp.float32)
    fc = jnp.dot(EC, wvC, preferred_element_type=jnp.float32)
    fl = jnp.dot(EL, wvL, preferred_element_type=jnp.float32)
    fc_row = jnp.tile(fc[:, 0].reshape(1, M), (1, N))
    fl_row = jnp.tile(fl[:, 0].reshape(1, M), (1, N))
    maskF = maskF_ref[...]
    w2C = jnp.dot(MEr * fc_row, maskF, preferred_element_type=jnp.float32)
    w2L = jnp.dot(MEr * fl_row, maskF, preferred_element_type=jnp.float32)
    uC = jnp.dot(Ht, aC[:D, :], preferred_element_type=jnp.float32)
    vC = jnp.dot(Ht, aC[D:2 * D, :], preferred_element_type=jnp.float32)
    uL = jnp.dot(Ht, aL[:D, :], preferred_element_type=jnp.float32)
    vL = jnp.dot(Ht, aL[D:2 * D, :], preferred_element_type=jnp.float32)
    cstC = jnp.dot(WECb, aC[2 * D:, :], preferred_element_type=jnp.float32)
    cstL = jnp.dot(WELb, aL[2 * D:, :], preferred_element_type=jnp.float32)

    adjN = AH_ref[0] > 0
    scC = _lrelu(uC + jnp.broadcast_to(vC[:, 0], (N, N)) + w2C + cstC)
    scL = _lrelu(uL + jnp.broadcast_to(vL[:, 0], (N, N)) + w2L + cstL)
    attnC = _softmax_rows(scC, adjN)
    attnL = _softmax_rows(scL, adjN)
    attnC_sc[...] = attnC
    attnL_sc[...] = attnL

    aggC = jnp.dot(attnC, Ht, preferred_element_type=jnp.float32)
    aggL = jnp.dot(attnL, Ht, preferred_element_type=jnp.float32)

    iota = jax.lax.broadcasted_iota(jnp.int32, (N, 1), 0)
    pm = jnp.zeros((N, 1), dtype=jnp.bool_)
    for k in range(8):
        pm = pm | (iota == pni_ref[k])
    any_adj = jnp.max(AH_ref[0], axis=1, keepdims=True) > 0
    rm = pm & any_adj
    Hn_ref[0] = jnp.where(rm, 0.5 * (aggC + aggL), Ht)

    Hm_ref[0] = jnp.zeros((N, D), dtype=jnp.float32)
    for c in copies:
        c.wait()
    for k in range(8):
        i = pni_ref[k]
        rME = rme_sc[k * N:(k + 1) * N, :]
        ECr = jnp.dot(rME, FC, preferred_element_type=jnp.float32) + WECb
        ELr = jnp.dot(rME, FL, preferred_element_type=jnp.float32) + WELb
        XC = ECr * Ht
        XL = ELr * Ht
        aCr = attnC_sc[pl.ds(i, 1), :]
        aLr = attnL_sc[pl.ds(i, 1), :]
        mk = 0.5 * (jnp.dot(aCr, XC, preferred_element_type=jnp.float32)
                    + jnp.dot(aLr, XL, preferred_element_type=jnp.float32))
        Hm_ref[0, pl.ds(i, 1), :] = mk
    Hm_ref[0] = Hm_ref[0] * rm.astype(jnp.float32)

    ECt = FC + WECb
    ELt = FL + WELb
    xC = jnp.dot(ECt, bC[:D, :], preferred_element_type=jnp.float32)
    yC = jnp.dot(ECt, bC[D:2 * D, :], preferred_element_type=jnp.float32)
    xL = jnp.dot(ELt, bL[:D, :], preferred_element_type=jnp.float32)
    yL = jnp.dot(ELt, bL[D:2 * D, :], preferred_element_type=jnp.float32)
    zC = jnp.dot(H[0:1, :], bC[2 * D:, :], preferred_element_type=jnp.float32)
    zL = jnp.dot(H[0:1, :], bL[2 * D:, :], preferred_element_type=jnp.float32)

    adjE = AE_ref[0] > 0
    sEC = _lrelu(xC + jnp.broadcast_to(yC[:, 0], (M, M)) + zC)
    sEL = _lrelu(xL + jnp.broadcast_to(yL[:, 0], (M, M)) + zL)
    attnEC = _softmax_rows(sEC, adjE)
    attnEL = _softmax_rows(sEL, adjE)
    aggEC = jnp.dot(attnEC, ECt, preferred_element_type=jnp.float32)
    aggEL = jnp.dot(attnEL, ELt, preferred_element_type=jnp.float32)

    iotaE = jax.lax.broadcasted_iota(jnp.int32, (M, 1), 0)
    pmE = jnp.zeros((M, 1), dtype=jnp.bool_)
    for k in range(8):
        pmE = pmE | (iotaE == pei_ref[k])
    any_adjE = jnp.max(AE_ref[0], axis=1, keepdims=True) > 0
    rmE = pmE & any_adjE
    ECn_ref[0] = jnp.where(rmE, aggEC, ECt)
    ELn_ref[0] = jnp.where(rmE, aggEL, ELt)


def kernel(H, E_C, E_L, AH, AE, ME, MH, path_node_indices, path_edge_indices,
           W_H_w, W_H_b, W_EC_w, W_EC_b, W_EL_w, W_EL_b, a_C, a_L, b_C, b_L):
    B, N, ND = H.shape
    M = E_C.shape[1]
    Fe = E_C.shape[2]
    D = W_H_w.shape[1]
    A3 = a_C.shape[0]
    f32 = jnp.float32

    MEr = ME.reshape(B, N, N * M)
    lidx = jnp.arange(N * M, dtype=jnp.int32)[:, None]
    maskF = (lidx // M == jnp.arange(N, dtype=jnp.int32)[None, :]).astype(f32)

    grid_spec = pltpu.PrefetchScalarGridSpec(
        num_scalar_prefetch=2,
        grid=(B,),
        in_specs=[
            pl.BlockSpec((1, N, ND), lambda b, pni, pei: (b, 0, 0)),
            pl.BlockSpec((1, M, Fe), lambda b, pni, pei: (b, 0, 0)),
            pl.BlockSpec((1, M, Fe), lambda b, pni, pei: (b, 0, 0)),
            pl.BlockSpec((1, N, N), lambda b, pni, pei: (0, 0, 0)),
            pl.BlockSpec((1, M, M), lambda b, pni, pei: (0, 0, 0)),
            pl.BlockSpec((1, N, N * M), lambda b, pni, pei: (b, 0, 0)),
            pl.BlockSpec(memory_space=pltpu.MemorySpace.HBM),
            pl.BlockSpec((N * M, N), lambda b, pni, pei: (0, 0)),
            pl.BlockSpec((ND, D), lambda b, pni, pei: (0, 0)),
            pl.BlockSpec((D,), lambda b, pni, pei: (0,)),
            pl.BlockSpec((Fe, D), lambda b, pni, pei: (0, 0)),
            pl.BlockSpec((D,), lambda b, pni, pei: (0,)),
            pl.BlockSpec((Fe, D), lambda b, pni, pei: (0, 0)),
            pl.BlockSpec((D,), lambda b, pni, pei: (0,)),
            pl.BlockSpec((A3, 1), lambda b, pni, pei: (0, 0)),
            pl.BlockSpec((A3, 1), lambda b, pni, pei: (0, 0)),
            pl.BlockSpec((A3, 1), lambda b, pni, pei: (0, 0)),
            pl.BlockSpec((A3, 1), lambda b, pni, pei: (0, 0)),
        ],
        out_specs=[
            pl.BlockSpec((1, N, D), lambda b, pni, pei: (b, 0, 0)),
            pl.BlockSpec((1, M, D), lambda b, pni, pei: (b, 0, 0)),
            pl.BlockSpec((1, M, D), lambda b, pni, pei: (b, 0, 0)),
            pl.BlockSpec((1, N, D), lambda b, pni, pei: (b, 0, 0)),
        ],
        scratch_shapes=[
            pltpu.VMEM((N, N), f32),
            pltpu.VMEM((N, N), f32),
            pltpu.VMEM((8 * N, M), f32),
            pltpu.SemaphoreType.DMA,
        ],
    )

    out_shape = [
        jax.ShapeDtypeStruct((B, N, D), f32),
        jax.ShapeDtypeStruct((B, M, D), f32),
        jax.ShapeDtypeStruct((B, M, D), f32),
        jax.ShapeDtypeStruct((B, N, D), f32),
    ]

    Hn, ECn, ELn, Hm = pl.pallas_call(
        _egat_kernel,
        grid_spec=grid_spec,
        out_shape=out_shape,
        compiler_params=pltpu.CompilerParams(
            dimension_semantics=("arbitrary",),
        ),
    )(path_node_indices, path_edge_indices,
      H, E_C, E_L, AH, AE, MEr, ME, maskF,
      W_H_w, W_H_b, W_EC_w, W_EC_b, W_EL_w, W_EL_b, a_C, a_L, b_C, b_L)

    return (Hn, ECn, ELn, Hm)


# speedup vs baseline: 1.3655x; 1.3655x over previous
import jax
import jax.numpy as jnp
from jax.experimental import pallas as pl
from jax.experimental.pallas import tpu as pltpu

_NEG = -1e30
_K = 8


def _lrelu(x):
    return jnp.where(x >= 0, x, 0.2 * x)


def _dotT(a, b):
    return jax.lax.dot_general(a, b, (((0,), (0,)), ((), ())),
                               preferred_element_type=jnp.float32)


def _softmax_cols(score, adj_bool):
    masked = jnp.where(adj_bool, score, _NEG)
    m = jnp.max(masked, axis=0, keepdims=True)
    e = jnp.exp(masked - m)
    return e / jnp.sum(e, axis=0, keepdims=True)


def _egat_kernel(pni_ref, pei_ref,
                 H_ref, EC_ref, EL_ref, AH_ref, AE_ref, MEany_ref,
                 WH_ref, WHb_ref, WEC_ref, WECb_ref, WEL_ref, WELb_ref,
                 aC_ref, aL_ref, bC_ref, bL_ref,
                 Hn_ref, ECn_ref, ELn_ref, Hm_ref,
                 rme_sc, dma_sem):
    N = 64
    M = 50
    D = 256
    b = pl.program_id(0)
    f32 = jnp.float32

    copies = []
    for k in range(_K):
        i = pni_ref[k]
        c = pltpu.make_async_copy(
            MEany_ref.at[b, pl.ds(i * N, N), :],
            rme_sc.at[pl.ds(k * N, N), :],
            dma_sem,
        )
        c.start()
        copies.append(c)

    H = H_ref[0]
    EC = EC_ref[0]
    EL = EL_ref[0]
    aC = aC_ref[...]
    aL = aL_ref[...]
    bC = bC_ref[...]
    bL = bL_ref[...]
    WECb = WECb_ref[...].reshape(1, D)
    WELb = WELb_ref[...].reshape(1, D)

    Ht = jnp.dot(H, WH_ref[...], preferred_element_type=f32) + WHb_ref[...].reshape(1, D)
    FC = jnp.dot(EC, WEC_ref[...], preferred_element_type=f32)
    FL = jnp.dot(EL, WEL_ref[...], preferred_element_type=f32)

    wvC = jnp.dot(WEC_ref[...], aC[2 * D:, :], preferred_element_type=f32)
    wvL = jnp.dot(WEL_ref[...], aL[2 * D:, :], preferred_element_type=f32)
    fc = jnp.dot(EC, wvC, preferred_element_type=f32)
    fl = jnp.dot(EL, wvL, preferred_element_type=f32)
    fcfl = jnp.concatenate([fc, fl], axis=1)
    uC = jnp.dot(Ht, aC[:D, :], preferred_element_type=f32)
    vC = jnp.dot(Ht, aC[D:2 * D, :], preferred_element_type=f32)
    uL = jnp.dot(Ht, aL[:D, :], preferred_element_type=f32)
    vL = jnp.dot(Ht, aL[D:2 * D, :], preferred_element_type=f32)
    cstC = jnp.dot(WECb, aC[2 * D:, :], preferred_element_type=f32)
    cstL = jnp.dot(WELb, aL[2 * D:, :], preferred_element_type=f32)

    iota = jax.lax.broadcasted_iota(jnp.int32, (N, 1), 0)
    i_row = jnp.concatenate(
        [jnp.full((1, 1), pni_ref[k], jnp.int32) for k in range(_K)], axis=1)
    onehot = (iota == i_row).astype(f32)

    AHf = (AH_ref[0] > 0).astype(f32)
    adjT8 = _dotT(AHf, onehot)
    any_adj = jnp.max(AHf, axis=1, keepdims=True)
    any8 = _dotT(any_adj, onehot)
    u8C = _dotT(uC, onehot)
    u8L = _dotT(uL, onehot)

    ECt = FC + WECb
    ELt = FL + WELb
    xC = jnp.dot(ECt, bC[:D, :], preferred_element_type=f32)
    yC = jnp.dot(ECt, bC[D:2 * D, :], preferred_element_type=f32)
    xL = jnp.dot(ELt, bL[:D, :], preferred_element_type=f32)
    yL = jnp.dot(ELt, bL[D:2 * D, :], preferred_element_type=f32)
    zC = jnp.dot(H[0:1, :], bC[2 * D:, :], preferred_element_type=f32)
    zL = jnp.dot(H[0:1, :], bL[2 * D:, :], preferred_element_type=f32)

    iotaE = jax.lax.broadcasted_iota(jnp.int32, (M, 1), 0)
    p_row = jnp.concatenate(
        [jnp.full((1, 1), pei_ref[k], jnp.int32) for k in range(_K)], axis=1)
    onehotE = (iotaE == p_row).astype(f32)
    AEf = (AE_ref[0] > 0).astype(f32)
    adjET8 = _dotT(AEf, onehotE)
    anyE = jnp.max(AEf, axis=1, keepdims=True)
    anyE8 = _dotT(anyE, onehotE)
    x8C = _dotT(xC, onehotE)
    x8L = _dotT(xL, onehotE)

    scTEC = _lrelu(yC + x8C + zC)
    scTEL = _lrelu(yL + x8L + zL)
    attnTEC = _softmax_cols(scTEC, adjET8 > 0.5)
    attnTEL = _softmax_cols(scTEL, adjET8 > 0.5)
    aggEC8 = _dotT(attnTEC, ECt)
    aggEL8 = _dotT(attnTEL, ELt)
    rowsEC8 = _dotT(onehotE, ECt)
    rowsEL8 = _dotT(onehotE, ELt)
    anyE8c = anyE8.reshape(_K, 1)
    valEC = jnp.where(anyE8c > 0.5, aggEC8, rowsEC8)
    valEL = jnp.where(anyE8c > 0.5, aggEL8, rowsEL8)

    ECn_ref[0] = ECt
    ELn_ref[0] = ELt
    for k in range(_K):
        p = pei_ref[k]
        ECn_ref[0, pl.ds(p, 1), :] = valEC[k:k + 1, :]
        ELn_ref[0, pl.ds(p, 1), :] = valEL[k:k + 1, :]

    for c in copies:
        c.wait()

    wcols_C = []
    wcols_L = []
    for k in range(_K):
        wk = jnp.dot(rme_sc[k * N:(k + 1) * N, :], fcfl,
                     preferred_element_type=f32)
        wcols_C.append(wk[:, 0:1])
        wcols_L.append(wk[:, 1:2])
    wTC = jnp.concatenate(wcols_C, axis=1)
    wTL = jnp.concatenate(wcols_L, axis=1)

    scTC = _lrelu(vC + u8C + wTC + cstC)
    scTL = _lrelu(vL + u8L + wTL + cstL)
    adjb = adjT8 > 0.5
    attnTC = _softmax_cols(scTC, adjb)
    attnTL = _softmax_cols(scTL, adjb)
    aggC8 = _dotT(attnTC, Ht)
    aggL8 = _dotT(attnTL, Ht)
    rowsH8 = _dotT(onehot, Ht)
    any8c = any8.reshape(_K, 1)
    valH = jnp.where(any8c > 0.5, 0.5 * (aggC8 + aggL8), rowsH8)

    Hn_ref[0] = Ht
    Hm_ref[0] = jnp.zeros((N, D), dtype=f32)
    for k in range(_K):
        i = pni_ref[k]
        rME = rme_sc[k * N:(k + 1) * N, :]
        ECr = jnp.dot(rME, FC, preferred_element_type=f32)
        ELr = jnp.dot(rME, FL, preferred_element_type=f32)
        mkC = _dotT(attnTC[:, k:k + 1], Ht * ECr) + WECb * aggC8[k:k + 1, :]
        mkL = _dotT(attnTL[:, k:k + 1], Ht * ELr) + WELb * aggL8[k:k + 1, :]
        mk = 0.5 * (mkC + mkL) * any8[0:1, k:k + 1]
        Hm_ref[0, pl.ds(i, 1), :] = mk
        Hn_ref[0, pl.ds(i, 1), :] = valH[k:k + 1, :]


def kernel(H, E_C, E_L, AH, AE, ME, MH, path_node_indices, path_edge_indices,
           W_H_w, W_H_b, W_EC_w, W_EC_b, W_EL_w, W_EL_b, a_C, a_L, b_C, b_L):
    B, N, ND = H.shape
    M = E_C.shape[1]
    Fe = E_C.shape[2]
    D = W_H_w.shape[1]
    A3 = a_C.shape[0]
    f32 = jnp.float32

    grid_spec = pltpu.PrefetchScalarGridSpec(
        num_scalar_prefetch=2,
        grid=(B,),
        in_specs=[
            pl.BlockSpec((1, N, ND), lambda b, pni, pei: (b, 0, 0)),
            pl.BlockSpec((1, M, Fe), lambda b, pni, pei: (b, 0, 0)),
            pl.BlockSpec((1, M, Fe), lambda b, pni, pei: (b, 0, 0)),
            pl.BlockSpec((1, N, N), lambda b, pni, pei: (0, 0, 0)),
            pl.BlockSpec((1, M, M), lambda b, pni, pei: (0, 0, 0)),
            pl.BlockSpec(memory_space=pltpu.MemorySpace.HBM),
            pl.BlockSpec((ND, D), lambda b, pni, pei: (0, 0)),
            pl.BlockSpec((D,), lambda b, pni, pei: (0,)),
            pl.BlockSpec((Fe, D), lambda b, pni, pei: (0, 0)),
            pl.BlockSpec((D,), lambda b, pni, pei: (0,)),
            pl.BlockSpec((Fe, D), lambda b, pni, pei: (0, 0)),
            pl.BlockSpec((D,), lambda b, pni, pei: (0,)),
            pl.BlockSpec((A3, 1), lambda b, pni, pei: (0, 0)),
            pl.BlockSpec((A3, 1), lambda b, pni, pei: (0, 0)),
            pl.BlockSpec((A3, 1), lambda b, pni, pei: (0, 0)),
            pl.BlockSpec((A3, 1), lambda b, pni, pei: (0, 0)),
        ],
        out_specs=[
            pl.BlockSpec((1, N, D), lambda b, pni, pei: (b, 0, 0)),
            pl.BlockSpec((1, M, D), lambda b, pni, pei: (b, 0, 0)),
            pl.BlockSpec((1, M, D), lambda b, pni, pei: (b, 0, 0)),
            pl.BlockSpec((1, N, D), lambda b, pni, pei: (b, 0, 0)),
        ],
        scratch_shapes=[
            pltpu.VMEM((_K * N, M), f32),
            pltpu.SemaphoreType.DMA,
        ],
    )

    out_shape = [
        jax.ShapeDtypeStruct((B, N, D), f32),
        jax.ShapeDtypeStruct((B, M, D), f32),
        jax.ShapeDtypeStruct((B, M, D), f32),
        jax.ShapeDtypeStruct((B, N, D), f32),
    ]

    Hn, ECn, ELn, Hm = pl.pallas_call(
        _egat_kernel,
        grid_spec=grid_spec,
        out_shape=out_shape,
        compiler_params=pltpu.CompilerParams(
            dimension_semantics=("arbitrary",),
        ),
    )(path_node_indices, path_edge_indices,
      H, E_C, E_L, AH, AE, ME,
      W_H_w, W_H_b, W_EC_w, W_EC_b, W_EL_w, W_EL_b, a_C, a_L, b_C, b_L)

    return (Hn, ECn, ELn, Hm)


# speedup vs baseline: 1.4997x; 1.0983x over previous
import jax
import jax.numpy as jnp
from jax.experimental import pallas as pl
from jax.experimental.pallas import tpu as pltpu

_NEG = -1e30
_K = 8


def _lrelu(x):
    return jnp.where(x >= 0, x, 0.2 * x)


def _dotT(a, b):
    return jax.lax.dot_general(a, b, (((0,), (0,)), ((), ())),
                               preferred_element_type=jnp.float32)


def _softmax_cols(score, adj_bool):
    masked = jnp.where(adj_bool, score, _NEG)
    m = jnp.max(masked, axis=0, keepdims=True)
    e = jnp.exp(masked - m)
    return e / jnp.sum(e, axis=0, keepdims=True)


def _egat_kernel(pni_ref, pei_ref,
                 H_ref, EC_ref, EL_ref, AH_ref, AE_ref, MEany_ref,
                 WH_ref, WHb_ref, WEC_ref, WECb_ref, WEL_ref, WELb_ref,
                 aC_ref, aL_ref, bC_ref, bL_ref,
                 Hn_ref, ECn_ref, ELn_ref, Hm_ref,
                 rme_sc, dma_sem):
    N = 64
    M = 50
    D = 256
    b = pl.program_id(0)
    nb = pl.num_programs(0)
    f32 = jnp.float32
    slot = jax.lax.rem(b, 2)

    def _issue(bb, s):
        for k in range(_K):
            i = pni_ref[k]
            pltpu.make_async_copy(
                MEany_ref.at[bb, pl.ds(i * N, N), :],
                rme_sc.at[s, pl.ds(k * N, N), :],
                dma_sem.at[s],
            ).start()

    @pl.when(b == 0)
    def _():
        _issue(0, 0)

    @pl.when(b + 1 < nb)
    def _():
        _issue(b + 1, jax.lax.rem(b + 1, 2))

    H = H_ref[0]
    EC = EC_ref[0]
    EL = EL_ref[0]
    aC = aC_ref[...]
    aL = aL_ref[...]
    bC = bC_ref[...]
    bL = bL_ref[...]
    WECb = WECb_ref[...].reshape(1, D)
    WELb = WELb_ref[...].reshape(1, D)

    Ht = jnp.dot(H, WH_ref[...], preferred_element_type=f32) + WHb_ref[...].reshape(1, D)
    FC = jnp.dot(EC, WEC_ref[...], preferred_element_type=f32)
    FL = jnp.dot(EL, WEL_ref[...], preferred_element_type=f32)

    wvC = jnp.dot(WEC_ref[...], aC[2 * D:, :], preferred_element_type=f32)
    wvL = jnp.dot(WEL_ref[...], aL[2 * D:, :], preferred_element_type=f32)
    fc = jnp.dot(EC, wvC, preferred_element_type=f32)
    fl = jnp.dot(EL, wvL, preferred_element_type=f32)
    fcfl = jnp.concatenate([fc, fl], axis=1)
    uC = jnp.dot(Ht, aC[:D, :], preferred_element_type=f32)
    vC = jnp.dot(Ht, aC[D:2 * D, :], preferred_element_type=f32)
    uL = jnp.dot(Ht, aL[:D, :], preferred_element_type=f32)
    vL = jnp.dot(Ht, aL[D:2 * D, :], preferred_element_type=f32)
    cstC = jnp.dot(WECb, aC[2 * D:, :], preferred_element_type=f32)
    cstL = jnp.dot(WELb, aL[2 * D:, :], preferred_element_type=f32)

    iota = jax.lax.broadcasted_iota(jnp.int32, (N, 1), 0)
    i_row = jnp.concatenate(
        [jnp.full((1, 1), pni_ref[k], jnp.int32) for k in range(_K)], axis=1)
    onehot = (iota == i_row).astype(f32)

    AHf = (AH_ref[0] > 0).astype(f32)
    adjT8 = _dotT(AHf, onehot)
    any_adj = jnp.max(AHf, axis=1, keepdims=True)
    any8 = _dotT(any_adj, onehot)
    u8C = _dotT(uC, onehot)
    u8L = _dotT(uL, onehot)

    ECt = FC + WECb
    ELt = FL + WELb
    xC = jnp.dot(ECt, bC[:D, :], preferred_element_type=f32)
    yC = jnp.dot(ECt, bC[D:2 * D, :], preferred_element_type=f32)
    xL = jnp.dot(ELt, bL[:D, :], preferred_element_type=f32)
    yL = jnp.dot(ELt, bL[D:2 * D, :], preferred_element_type=f32)
    zC = jnp.dot(H[0:1, :], bC[2 * D:, :], preferred_element_type=f32)
    zL = jnp.dot(H[0:1, :], bL[2 * D:, :], preferred_element_type=f32)

    iotaE = jax.lax.broadcasted_iota(jnp.int32, (M, 1), 0)
    p_row = jnp.concatenate(
        [jnp.full((1, 1), pei_ref[k], jnp.int32) for k in range(_K)], axis=1)
    onehotE = (iotaE == p_row).astype(f32)
    AEf = (AE_ref[0] > 0).astype(f32)
    adjET8 = _dotT(AEf, onehotE)
    anyE = jnp.max(AEf, axis=1, keepdims=True)
    anyE8 = _dotT(anyE, onehotE)
    x8C = _dotT(xC, onehotE)
    x8L = _dotT(xL, onehotE)

    scTEC = _lrelu(yC + x8C + zC)
    scTEL = _lrelu(yL + x8L + zL)
    attnTEC = _softmax_cols(scTEC, adjET8 > 0.5)
    attnTEL = _softmax_cols(scTEL, adjET8 > 0.5)
    aggEC8 = _dotT(attnTEC, ECt)
    aggEL8 = _dotT(attnTEL, ELt)
    rowsEC8 = _dotT(onehotE, ECt)
    rowsEL8 = _dotT(onehotE, ELt)
    anyE8c = anyE8.reshape(_K, 1)
    valEC = jnp.where(anyE8c > 0.5, aggEC8, rowsEC8)
    valEL = jnp.where(anyE8c > 0.5, aggEL8, rowsEL8)

    ECn_ref[0] = ECt
    ELn_ref[0] = ELt
    for k in range(_K):
        p = pei_ref[k]
        ECn_ref[0, pl.ds(p, 1), :] = valEC[k:k + 1, :]
        ELn_ref[0, pl.ds(p, 1), :] = valEL[k:k + 1, :]

    for k in range(_K):
        i = pni_ref[k]
        pltpu.make_async_copy(
            MEany_ref.at[b, pl.ds(i * N, N), :],
            rme_sc.at[slot, pl.ds(k * N, N), :],
            dma_sem.at[slot],
        ).wait()

    wcols_C = []
    wcols_L = []
    for k in range(_K):
        wk = jnp.dot(rme_sc[slot, pl.ds(k * N, N), :], fcfl,
                     preferred_element_type=f32)
        wcols_C.append(wk[:, 0:1])
        wcols_L.append(wk[:, 1:2])
    wTC = jnp.concatenate(wcols_C, axis=1)
    wTL = jnp.concatenate(wcols_L, axis=1)

    scTC = _lrelu(vC + u8C + wTC + cstC)
    scTL = _lrelu(vL + u8L + wTL + cstL)
    adjb = adjT8 > 0.5
    attnTC = _softmax_cols(scTC, adjb)
    attnTL = _softmax_cols(scTL, adjb)
    aggC8 = _dotT(attnTC, Ht)
    aggL8 = _dotT(attnTL, Ht)
    rowsH8 = _dotT(onehot, Ht)
    any8c = any8.reshape(_K, 1)
    valH = jnp.where(any8c > 0.5, 0.5 * (aggC8 + aggL8), rowsH8)

    Hn_ref[0] = Ht
    Hm_ref[0] = jnp.zeros((N, D), dtype=f32)
    for k in range(_K):
        i = pni_ref[k]
        rME = rme_sc[slot, pl.ds(k * N, N), :]
        ECr = jnp.dot(rME, FC, preferred_element_type=f32)
        ELr = jnp.dot(rME, FL, preferred_element_type=f32)
        mkC = _dotT(attnTC[:, k:k + 1], Ht * ECr) + WECb * aggC8[k:k + 1, :]
        mkL = _dotT(attnTL[:, k:k + 1], Ht * ELr) + WELb * aggL8[k:k + 1, :]
        mk = 0.5 * (mkC + mkL) * any8[0:1, k:k + 1]
        Hm_ref[0, pl.ds(i, 1), :] = mk
        Hn_ref[0, pl.ds(i, 1), :] = valH[k:k + 1, :]


def kernel(H, E_C, E_L, AH, AE, ME, MH, path_node_indices, path_edge_indices,
           W_H_w, W_H_b, W_EC_w, W_EC_b, W_EL_w, W_EL_b, a_C, a_L, b_C, b_L):
    B, N, ND = H.shape
    M = E_C.shape[1]
    Fe = E_C.shape[2]
    D = W_H_w.shape[1]
    A3 = a_C.shape[0]
    f32 = jnp.float32

    grid_spec = pltpu.PrefetchScalarGridSpec(
        num_scalar_prefetch=2,
        grid=(B,),
        in_specs=[
            pl.BlockSpec((1, N, ND), lambda b, pni, pei: (b, 0, 0)),
            pl.BlockSpec((1, M, Fe), lambda b, pni, pei: (b, 0, 0)),
            pl.BlockSpec((1, M, Fe), lambda b, pni, pei: (b, 0, 0)),
            pl.BlockSpec((1, N, N), lambda b, pni, pei: (0, 0, 0)),
            pl.BlockSpec((1, M, M), lambda b, pni, pei: (0, 0, 0)),
            pl.BlockSpec(memory_space=pltpu.MemorySpace.HBM),
            pl.BlockSpec((ND, D), lambda b, pni, pei: (0, 0)),
            pl.BlockSpec((D,), lambda b, pni, pei: (0,)),
            pl.BlockSpec((Fe, D), lambda b, pni, pei: (0, 0)),
            pl.BlockSpec((D,), lambda b, pni, pei: (0,)),
            pl.BlockSpec((Fe, D), lambda b, pni, pei: (0, 0)),
            pl.BlockSpec((D,), lambda b, pni, pei: (0,)),
            pl.BlockSpec((A3, 1), lambda b, pni, pei: (0, 0)),
            pl.BlockSpec((A3, 1), lambda b, pni, pei: (0, 0)),
            pl.BlockSpec((A3, 1), lambda b, pni, pei: (0, 0)),
            pl.BlockSpec((A3, 1), lambda b, pni, pei: (0, 0)),
        ],
        out_specs=[
            pl.BlockSpec((1, N, D), lambda b, pni, pei: (b, 0, 0)),
            pl.BlockSpec((1, M, D), lambda b, pni, pei: (b, 0, 0)),
            pl.BlockSpec((1, M, D), lambda b, pni, pei: (b, 0, 0)),
            pl.BlockSpec((1, N, D), lambda b, pni, pei: (b, 0, 0)),
        ],
        scratch_shapes=[
            pltpu.VMEM((2, _K * N, M), f32),
            pltpu.SemaphoreType.DMA((2,)),
        ],
    )

    out_shape = [
        jax.ShapeDtypeStruct((B, N, D), f32),
        jax.ShapeDtypeStruct((B, M, D), f32),
        jax.ShapeDtypeStruct((B, M, D), f32),
        jax.ShapeDtypeStruct((B, N, D), f32),
    ]

    Hn, ECn, ELn, Hm = pl.pallas_call(
        _egat_kernel,
        grid_spec=grid_spec,
        out_shape=out_shape,
        compiler_params=pltpu.CompilerParams(
            dimension_semantics=("arbitrary",),
        ),
    )(path_node_indices, path_edge_indices,
      H, E_C, E_L, AH, AE, ME,
      W_H_w, W_H_b, W_EC_w, W_EC_b, W_EL_w, W_EL_b, a_C, a_L, b_C, b_L)

    return (Hn, ECn, ELn, Hm)


# speedup vs baseline: 2.4010x; 1.6009x over previous
import jax
import jax.numpy as jnp
from jax.experimental import pallas as pl
from jax.experimental.pallas import tpu as pltpu

_NEG = -1e30
_K = 8


def _lrelu(x):
    return jnp.where(x >= 0, x, 0.2 * x)


def _dotT(a, b):
    return jax.lax.dot_general(a, b, (((0,), (0,)), ((), ())),
                               preferred_element_type=jnp.float32)


def _dotNT(a, b):
    return jax.lax.dot_general(a, b, (((1,), (1,)), ((), ())),
                               preferred_element_type=jnp.float32)


def _softmax_rows(score, adj_bool):
    masked = jnp.where(adj_bool, score, _NEG)
    m = jnp.max(masked, axis=1, keepdims=True)
    e = jnp.exp(masked - m)
    return e / jnp.sum(e, axis=1, keepdims=True)


def _egat_kernel(pni_ref, pei_ref,
                 H_ref, ECt_ref, ELt_ref, AH_ref, AEt_ref, MEt_ref, maskMM_ref,
                 WH_ref, WHb_ref, WEC_ref, WECb_ref, WEL_ref, WELb_ref,
                 aCr_ref, aLr_ref, bCr_ref, bLr_ref,
                 Hn_ref, ECn_ref, ELn_ref, Hm_ref,
                 rme_sc, dma_sem):
    N = 64
    M = 50
    D = 256
    B_ = 8
    Fe_ = 128
    b = pl.program_id(0)
    f32 = jnp.float32

    def _copies():
        for k in range(_K):
            icol = pni_ref[k] // 2
            yield pltpu.make_async_copy(
                MEt_ref.at[:, :, pl.ds(icol * 128, 128)],
                rme_sc.at[:, :, pl.ds(k * 128, 128)],
                dma_sem,
            )

    @pl.when(b == 0)
    def _():
        for c in _copies():
            c.start()

    H = H_ref[0]

    onehotB = (jax.lax.broadcasted_iota(jnp.int32, (B_, 1), 0) == b).astype(f32)
    selB = maskMM_ref[...] * jnp.broadcast_to(
        onehotB[None, :, :], (M, B_, 1)).reshape(M * B_, 1)
    ECall = ECt_ref[...].reshape(M * B_, Fe_)
    ELall = ELt_ref[...].reshape(M * B_, Fe_)
    EC = _dotT(selB, ECall)
    EL = _dotT(selB, ELall)
    aCr = aCr_ref[...]
    aLr = aLr_ref[...]
    bCr = bCr_ref[...]
    bLr = bLr_ref[...]
    WECb = WECb_ref[...].reshape(1, D)
    WELb = WELb_ref[...].reshape(1, D)

    Ht = jnp.dot(H, WH_ref[...], preferred_element_type=f32) + WHb_ref[...].reshape(1, D)
    FC = jnp.dot(EC, WEC_ref[...], preferred_element_type=f32)
    FL = jnp.dot(EL, WEL_ref[...], preferred_element_type=f32)

    wvC = _dotNT(WEC_ref[...], aCr[:, 2 * D:])
    wvL = _dotNT(WEL_ref[...], aLr[:, 2 * D:])
    fc = jnp.dot(EC, wvC, preferred_element_type=f32)
    fl = jnp.dot(EL, wvL, preferred_element_type=f32)
    fcfl = jnp.concatenate([fc, fl], axis=1)
    uC = _dotNT(Ht, aCr[:, :D])
    vC = _dotNT(Ht, aCr[:, D:2 * D])
    uL = _dotNT(Ht, aLr[:, :D])
    vL = _dotNT(Ht, aLr[:, D:2 * D])
    cstC = _dotNT(WECb, aCr[:, 2 * D:])
    cstL = _dotNT(WELb, aLr[:, 2 * D:])

    iota = jax.lax.broadcasted_iota(jnp.int32, (N, 1), 0)
    i_row = jnp.concatenate(
        [jnp.full((1, 1), pni_ref[k], jnp.int32) for k in range(_K)], axis=1)
    onehot = (iota == i_row).astype(f32)

    AHf = (AH_ref[0] > 0).astype(f32)
    adj8 = _dotT(onehot, AHf)
    any_adj = jnp.max(AHf, axis=1, keepdims=True)
    any8 = _dotT(onehot, any_adj)
    u8C = _dotT(onehot, uC)
    u8L = _dotT(onehot, uL)
    v_rowC = vC[:, 0].reshape(1, N)
    v_rowL = vL[:, 0].reshape(1, N)

    ECt = FC + WECb
    ELt = FL + WELb
    xC = _dotNT(ECt, bCr[:, :D])
    yC = _dotNT(ECt, bCr[:, D:2 * D])
    xL = _dotNT(ELt, bLr[:, :D])
    yL = _dotNT(ELt, bLr[:, D:2 * D])
    zC = _dotNT(H[0:1, :], bCr[:, 2 * D:])
    zL = _dotNT(H[0:1, :], bLr[:, 2 * D:])

    iotaE = jax.lax.broadcasted_iota(jnp.int32, (M, 1), 0)
    p_row = jnp.concatenate(
        [jnp.full((1, 1), pei_ref[k], jnp.int32) for k in range(_K)], axis=1)
    onehotE = (iotaE == p_row).astype(f32)
    AEf = (AEt_ref[:, 0, :] > 0).astype(f32)
    adjE8 = _dotT(onehotE, AEf)
    anyE = jnp.max(AEf, axis=1, keepdims=True)
    anyE8 = _dotT(onehotE, anyE)
    x8C = _dotT(onehotE, xC)
    x8L = _dotT(onehotE, xL)
    y_rowC = yC[:, 0].reshape(1, M)
    y_rowL = yL[:, 0].reshape(1, M)

    sc8EC = _lrelu(x8C + y_rowC + zC)
    sc8EL = _lrelu(x8L + y_rowL + zL)
    attn8EC = _softmax_rows(sc8EC, adjE8 > 0.5)
    attn8EL = _softmax_rows(sc8EL, adjE8 > 0.5)
    aggEC8 = jnp.dot(attn8EC, ECt, preferred_element_type=f32)
    aggEL8 = jnp.dot(attn8EL, ELt, preferred_element_type=f32)
    rowsEC8 = _dotT(onehotE, ECt)
    rowsEL8 = _dotT(onehotE, ELt)
    valEC = jnp.where(anyE8 > 0.5, aggEC8, rowsEC8)
    valEL = jnp.where(anyE8 > 0.5, aggEL8, rowsEL8)

    ECn_ref[0] = ECt
    ELn_ref[0] = ELt
    for k in range(_K):
        p = pei_ref[k]
        ECn_ref[0, pl.ds(p, 1), :] = valEC[k:k + 1, :]
        ELn_ref[0, pl.ds(p, 1), :] = valEL[k:k + 1, :]

    @pl.when(b == 0)
    def _():
        for c in _copies():
            c.wait()

    odd = [jax.lax.rem(pni_ref[k], 2) == 1 for k in range(_K)]
    rme_all = rme_sc[...].reshape(M * B_, _K * 128)
    blkball = _dotT(selB, rme_all)
    blks = [blkball[:, k * 128:(k + 1) * 128] for k in range(_K)]
    w_rows_C = []
    w_rows_L = []
    for k in range(_K):
        wk = _dotT(fcfl, blks[k])
        w_rows_C.append(jnp.where(odd[k], wk[0:1, N:], wk[0:1, :N]))
        w_rows_L.append(jnp.where(odd[k], wk[1:2, N:], wk[1:2, :N]))
    w8C = jnp.concatenate(w_rows_C, axis=0)
    w8L = jnp.concatenate(w_rows_L, axis=0)

    sc8C = _lrelu(u8C + v_rowC + w8C + cstC)
    sc8L = _lrelu(u8L + v_rowL + w8L + cstL)
    adjb = adj8 > 0.5
    attn8C = _softmax_rows(sc8C, adjb)
    attn8L = _softmax_rows(sc8L, adjb)
    aggC8 = jnp.dot(attn8C, Ht, preferred_element_type=f32)
    aggL8 = jnp.dot(attn8L, Ht, preferred_element_type=f32)
    rowsH8 = _dotT(onehot, Ht)
    valH = jnp.where(any8 > 0.5, 0.5 * (aggC8 + aggL8), rowsH8)

    Hn_ref[0] = Ht
    Hm_ref[0] = jnp.zeros((N, D), dtype=f32)
    for k in range(_K):
        i = pni_ref[k]
        ECr2 = _dotT(blks[k], FC)
        ELr2 = _dotT(blks[k], FL)
        ECr = jnp.where(odd[k], ECr2[N:, :], ECr2[:N, :])
        ELr = jnp.where(odd[k], ELr2[N:, :], ELr2[:N, :])
        mkC = jnp.dot(attn8C[k:k + 1, :], Ht * ECr,
                      preferred_element_type=f32) + WECb * aggC8[k:k + 1, :]
        mkL = jnp.dot(attn8L[k:k + 1, :], Ht * ELr,
                      preferred_element_type=f32) + WELb * aggL8[k:k + 1, :]
        mk = 0.5 * (mkC + mkL) * any8[k:k + 1, 0:1]
        Hm_ref[0, pl.ds(i, 1), :] = mk
        Hn_ref[0, pl.ds(i, 1), :] = valH[k:k + 1, :]


def kernel(H, E_C, E_L, AH, AE, ME, MH, path_node_indices, path_edge_indices,
           W_H_w, W_H_b, W_EC_w, W_EC_b, W_EL_w, W_EL_b, a_C, a_L, b_C, b_L):
    B, N, ND = H.shape
    M = E_C.shape[1]
    Fe = E_C.shape[2]
    D = W_H_w.shape[1]
    A3 = a_C.shape[0]
    f32 = jnp.float32

    ECtv = jnp.transpose(E_C, (1, 0, 2))
    ELtv = jnp.transpose(E_L, (1, 0, 2))
    AEtv = jnp.transpose(AE, (1, 0, 2))
    MEtv = jnp.transpose(ME, (2, 0, 1))
    maskMM = (jnp.arange(M * B, dtype=jnp.int32)[:, None] // B
              == jnp.arange(M, dtype=jnp.int32)[None, :]).astype(f32)
    aCr = jnp.transpose(a_C, (1, 0))
    aLr = jnp.transpose(a_L, (1, 0))
    bCr = jnp.transpose(b_C, (1, 0))
    bLr = jnp.transpose(b_L, (1, 0))

    grid_spec = pltpu.PrefetchScalarGridSpec(
        num_scalar_prefetch=2,
        grid=(B,),
        in_specs=[
            pl.BlockSpec((1, N, ND), lambda b, pni, pei: (b, 0, 0)),
            pl.BlockSpec((M, B, Fe), lambda b, pni, pei: (0, 0, 0)),
            pl.BlockSpec((M, B, Fe), lambda b, pni, pei: (0, 0, 0)),
            pl.BlockSpec((1, N, N), lambda b, pni, pei: (0, 0, 0)),
            pl.BlockSpec((M, B, M), lambda b, pni, pei: (0, 0, 0)),
            pl.BlockSpec(memory_space=pltpu.MemorySpace.HBM),
            pl.BlockSpec((M * B, M), lambda b, pni, pei: (0, 0)),
            pl.BlockSpec((ND, D), lambda b, pni, pei: (0, 0)),
            pl.BlockSpec((D,), lambda b, pni, pei: (0,)),
            pl.BlockSpec((Fe, D), lambda b, pni, pei: (0, 0)),
            pl.BlockSpec((D,), lambda b, pni, pei: (0,)),
            pl.BlockSpec((Fe, D), lambda b, pni, pei: (0, 0)),
            pl.BlockSpec((D,), lambda b, pni, pei: (0,)),
            pl.BlockSpec((1, A3), lambda b, pni, pei: (0, 0)),
            pl.BlockSpec((1, A3), lambda b, pni, pei: (0, 0)),
            pl.BlockSpec((1, A3), lambda b, pni, pei: (0, 0)),
            pl.BlockSpec((1, A3), lambda b, pni, pei: (0, 0)),
        ],
        out_specs=[
            pl.BlockSpec((1, N, D), lambda b, pni, pei: (b, 0, 0)),
            pl.BlockSpec((1, M, D), lambda b, pni, pei: (b, 0, 0)),
            pl.BlockSpec((1, M, D), lambda b, pni, pei: (b, 0, 0)),
            pl.BlockSpec((1, N, D), lambda b, pni, pei: (b, 0, 0)),
        ],
        scratch_shapes=[
            pltpu.VMEM((M, B, _K * 128), f32),
            pltpu.SemaphoreType.DMA,
        ],
    )

    out_shape = [
        jax.ShapeDtypeStruct((B, N, D), f32),
        jax.ShapeDtypeStruct((B, M, D), f32),
        jax.ShapeDtypeStruct((B, M, D), f32),
        jax.ShapeDtypeStruct((B, N, D), f32),
    ]

    Hn, ECn, ELn, Hm = pl.pallas_call(
        _egat_kernel,
        grid_spec=grid_spec,
        out_shape=out_shape,
        compiler_params=pltpu.CompilerParams(
            dimension_semantics=("arbitrary",),
        ),
    )(path_node_indices, path_edge_indices,
      H, ECtv, ELtv, AH, AEtv, MEtv, maskMM,
      W_H_w, W_H_b, W_EC_w, W_EC_b, W_EL_w, W_EL_b, aCr, aLr, bCr, bLr)

    return (Hn, ECn, ELn, Hm)


# speedup vs baseline: 3.0290x; 1.2616x over previous
import jax
import jax.numpy as jnp
import numpy as np
from jax.experimental import pallas as pl
from jax.experimental.pallas import tpu as pltpu

_NEG = -1e30
_K = 8


def _lrelu(x):
    return jnp.where(x >= 0, x, 0.2 * x)


def _dotT(a, b):
    return jax.lax.dot_general(a, b, (((0,), (0,)), ((), ())),
                               preferred_element_type=jnp.float32)


def _dotNT(a, b):
    return jax.lax.dot_general(a, b, (((1,), (1,)), ((), ())),
                               preferred_element_type=jnp.float32)


def _softmax_rows(score, adj_bool):
    masked = jnp.where(adj_bool, score, _NEG)
    m = jnp.max(masked, axis=1, keepdims=True)
    e = jnp.exp(masked - m)
    return e / jnp.sum(e, axis=1, keepdims=True)


def _egat_kernel(pni_ref, pei_ref,
                 H_ref, ECt_ref, ELt_ref, AH_ref, AEt_ref, MEt_ref, maskMM_ref,
                 WH_ref, WHb_ref, WEC_ref, WECb_ref, WEL_ref, WELb_ref,
                 aCr_ref, aLr_ref, bCr_ref, bLr_ref,
                 Hn_ref, ECn_ref, ELn_ref, Hm_ref,
                 rme_sc, dma_sem):
    N = 64
    M = 50
    D = 256
    B_ = 8
    Fe_ = 128
    b = pl.program_id(0)
    f32 = jnp.float32

    def _copies():
        for k in range(_K):
            icol = pni_ref[k] // 2
            yield pltpu.make_async_copy(
                MEt_ref.at[:, :, pl.ds(icol * 128, 128)],
                rme_sc.at[:, :, pl.ds(k * 128, 128)],
                dma_sem,
            )

    @pl.when(b == 0)
    def _():
        for c in _copies():
            c.start()

    H = H_ref[0]

    onehotB = (jax.lax.broadcasted_iota(jnp.int32, (B_, 1), 0) == b).astype(f32)
    selB = maskMM_ref[...] * jnp.broadcast_to(
        onehotB[None, :, :], (M, B_, 1)).reshape(M * B_, 1)
    ECall = ECt_ref[...].reshape(M * B_, Fe_)
    ELall = ELt_ref[...].reshape(M * B_, Fe_)
    EC = _dotT(selB, ECall)
    EL = _dotT(selB, ELall)
    aCr = aCr_ref[...]
    aLr = aLr_ref[...]
    bCr = bCr_ref[...]
    bLr = bLr_ref[...]
    WECb = WECb_ref[...].reshape(1, D)
    WELb = WELb_ref[...].reshape(1, D)

    Ht = jnp.dot(H, WH_ref[...], preferred_element_type=f32) + WHb_ref[...].reshape(1, D)
    FC = jnp.dot(EC, WEC_ref[...], preferred_element_type=f32)
    FL = jnp.dot(EL, WEL_ref[...], preferred_element_type=f32)

    wvC = _dotNT(WEC_ref[...], aCr[:, 2 * D:])
    wvL = _dotNT(WEL_ref[...], aLr[:, 2 * D:])
    fc = jnp.dot(EC, wvC, preferred_element_type=f32)
    fl = jnp.dot(EL, wvL, preferred_element_type=f32)
    fcfl = jnp.concatenate([fc, fl], axis=1)
    uC = _dotNT(Ht, aCr[:, :D])
    vC = _dotNT(Ht, aCr[:, D:2 * D])
    uL = _dotNT(Ht, aLr[:, :D])
    vL = _dotNT(Ht, aLr[:, D:2 * D])
    cstC = _dotNT(WECb, aCr[:, 2 * D:])
    cstL = _dotNT(WELb, aLr[:, 2 * D:])

    iota = jax.lax.broadcasted_iota(jnp.int32, (N, 1), 0)
    i_row = jnp.concatenate(
        [jnp.full((1, 1), pni_ref[k], jnp.int32) for k in range(_K)], axis=1)
    onehot = (iota == i_row).astype(f32)

    AHf = (AH_ref[0] > 0).astype(f32)
    adj8 = _dotT(onehot, AHf)
    any_adj = jnp.max(AHf, axis=1, keepdims=True)
    any8 = _dotT(onehot, any_adj)
    u8C = _dotT(onehot, uC)
    u8L = _dotT(onehot, uL)
    v_rowC = vC[:, 0].reshape(1, N)
    v_rowL = vL[:, 0].reshape(1, N)

    ECt = FC + WECb
    ELt = FL + WELb
    xC = _dotNT(ECt, bCr[:, :D])
    yC = _dotNT(ECt, bCr[:, D:2 * D])
    xL = _dotNT(ELt, bLr[:, :D])
    yL = _dotNT(ELt, bLr[:, D:2 * D])
    zC = _dotNT(H[0:1, :], bCr[:, 2 * D:])
    zL = _dotNT(H[0:1, :], bLr[:, 2 * D:])

    iotaE = jax.lax.broadcasted_iota(jnp.int32, (M, 1), 0)
    p_row = jnp.concatenate(
        [jnp.full((1, 1), pei_ref[k], jnp.int32) for k in range(_K)], axis=1)
    onehotE = (iotaE == p_row).astype(f32)
    AEf = (AEt_ref[:, 0, :] > 0).astype(f32)
    adjE8 = _dotT(onehotE, AEf)
    anyE = jnp.max(AEf, axis=1, keepdims=True)
    anyE8 = _dotT(onehotE, anyE)
    x8C = _dotT(onehotE, xC)
    x8L = _dotT(onehotE, xL)
    y_rowC = yC[:, 0].reshape(1, M)
    y_rowL = yL[:, 0].reshape(1, M)

    sc8EC = _lrelu(x8C + y_rowC + zC)
    sc8EL = _lrelu(x8L + y_rowL + zL)
    attn8EC = _softmax_rows(sc8EC, adjE8 > 0.5)
    attn8EL = _softmax_rows(sc8EL, adjE8 > 0.5)
    aggEC8 = jnp.dot(attn8EC, ECt, preferred_element_type=f32)
    aggEL8 = jnp.dot(attn8EL, ELt, preferred_element_type=f32)
    rowsEC8 = _dotT(onehotE, ECt)
    rowsEL8 = _dotT(onehotE, ELt)
    valEC = jnp.where(anyE8 > 0.5, aggEC8, rowsEC8)
    valEL = jnp.where(anyE8 > 0.5, aggEL8, rowsEL8)

    ECn_ref[0] = ECt
    ELn_ref[0] = ELt
    for k in range(_K):
        p = pei_ref[k]
        ECn_ref[0, pl.ds(p, 1), :] = valEC[k:k + 1, :]
        ELn_ref[0, pl.ds(p, 1), :] = valEL[k:k + 1, :]

    @pl.when(b == 0)
    def _():
        for c in _copies():
            c.wait()

    odd = [jax.lax.rem(pni_ref[k], 2) == 1 for k in range(_K)]
    rme_all = rme_sc[...].reshape(M * B_, _K * 128)
    blkball = _dotT(selB, rme_all)
    blks = [blkball[:, k * 128:(k + 1) * 128] for k in range(_K)]
    w_rows_C = []
    w_rows_L = []
    for k in range(_K):
        wk = _dotT(fcfl, blks[k])
        w_rows_C.append(jnp.where(odd[k], wk[0:1, N:], wk[0:1, :N]))
        w_rows_L.append(jnp.where(odd[k], wk[1:2, N:], wk[1:2, :N]))
    w8C = jnp.concatenate(w_rows_C, axis=0)
    w8L = jnp.concatenate(w_rows_L, axis=0)

    sc8C = _lrelu(u8C + v_rowC + w8C + cstC)
    sc8L = _lrelu(u8L + v_rowL + w8L + cstL)
    adjb = adj8 > 0.5
    attn8C = _softmax_rows(sc8C, adjb)
    attn8L = _softmax_rows(sc8L, adjb)
    aggC8 = jnp.dot(attn8C, Ht, preferred_element_type=f32)
    aggL8 = jnp.dot(attn8L, Ht, preferred_element_type=f32)
    rowsH8 = _dotT(onehot, Ht)
    valH = jnp.where(any8 > 0.5, 0.5 * (aggC8 + aggL8), rowsH8)

    ECr2all = _dotT(blkball, FC)
    ELr2all = _dotT(blkball, FL)
    ECr_sel = jnp.concatenate(
        [jnp.where(odd[k], ECr2all[k * 128 + N:(k + 1) * 128, :],
                   ECr2all[k * 128:k * 128 + N, :]) for k in range(_K)], axis=0)
    ELr_sel = jnp.concatenate(
        [jnp.where(odd[k], ELr2all[k * 128 + N:(k + 1) * 128, :],
                   ELr2all[k * 128:k * 128 + N, :]) for k in range(_K)], axis=0)
    Ht_tiled = jnp.broadcast_to(Ht[None, :, :], (_K, N, D)).reshape(_K * N, D)
    XC = Ht_tiled * ECr_sel
    XL = Ht_tiled * ELr_sel
    bd = (jax.lax.broadcasted_iota(jnp.int32, (_K, _K * N), 1) // N
          == jax.lax.broadcasted_iota(jnp.int32, (_K, _K * N), 0)).astype(f32)
    PC = jnp.tile(attn8C, (1, _K)) * bd
    PL = jnp.tile(attn8L, (1, _K)) * bd
    mkC8 = jnp.dot(PC, XC, preferred_element_type=f32) + WECb * aggC8
    mkL8 = jnp.dot(PL, XL, preferred_element_type=f32) + WELb * aggL8
    mk8 = 0.5 * (mkC8 + mkL8) * any8

    Hn_ref[0] = Ht
    Hm_ref[0] = jnp.zeros((N, D), dtype=f32)
    for k in range(_K):
        i = pni_ref[k]
        Hm_ref[0, pl.ds(i, 1), :] = mk8[k:k + 1, :]
        Hn_ref[0, pl.ds(i, 1), :] = valH[k:k + 1, :]


def kernel(H, E_C, E_L, AH, AE, ME, MH, path_node_indices, path_edge_indices,
           W_H_w, W_H_b, W_EC_w, W_EC_b, W_EL_w, W_EL_b, a_C, a_L, b_C, b_L):
    B, N, ND = H.shape
    M = E_C.shape[1]
    Fe = E_C.shape[2]
    D = W_H_w.shape[1]
    A3 = a_C.shape[0]
    f32 = jnp.float32

    ECtv = jnp.transpose(E_C, (1, 0, 2))
    ELtv = jnp.transpose(E_L, (1, 0, 2))
    AEtv = jnp.transpose(AE, (1, 0, 2))
    MEtv = jnp.transpose(ME, (2, 0, 1))
    maskMM = jnp.asarray((np.arange(M * B)[:, None] // B
                          == np.arange(M)[None, :]).astype(np.float32))
    aCr = jnp.transpose(a_C, (1, 0))
    aLr = jnp.transpose(a_L, (1, 0))
    bCr = jnp.transpose(b_C, (1, 0))
    bLr = jnp.transpose(b_L, (1, 0))

    grid_spec = pltpu.PrefetchScalarGridSpec(
        num_scalar_prefetch=2,
        grid=(B,),
        in_specs=[
            pl.BlockSpec((1, N, ND), lambda b, pni, pei: (b, 0, 0)),
            pl.BlockSpec((M, B, Fe), lambda b, pni, pei: (0, 0, 0)),
            pl.BlockSpec((M, B, Fe), lambda b, pni, pei: (0, 0, 0)),
            pl.BlockSpec((1, N, N), lambda b, pni, pei: (0, 0, 0)),
            pl.BlockSpec((M, B, M), lambda b, pni, pei: (0, 0, 0)),
            pl.BlockSpec(memory_space=pltpu.MemorySpace.HBM),
            pl.BlockSpec((M * B, M), lambda b, pni, pei: (0, 0)),
            pl.BlockSpec((ND, D), lambda b, pni, pei: (0, 0)),
            pl.BlockSpec((D,), lambda b, pni, pei: (0,)),
            pl.BlockSpec((Fe, D), lambda b, pni, pei: (0, 0)),
            pl.BlockSpec((D,), lambda b, pni, pei: (0,)),
            pl.BlockSpec((Fe, D), lambda b, pni, pei: (0, 0)),
            pl.BlockSpec((D,), lambda b, pni, pei: (0,)),
            pl.BlockSpec((1, A3), lambda b, pni, pei: (0, 0)),
            pl.BlockSpec((1, A3), lambda b, pni, pei: (0, 0)),
            pl.BlockSpec((1, A3), lambda b, pni, pei: (0, 0)),
            pl.BlockSpec((1, A3), lambda b, pni, pei: (0, 0)),
        ],
        out_specs=[
            pl.BlockSpec((1, N, D), lambda b, pni, pei: (b, 0, 0)),
            pl.BlockSpec((1, M, D), lambda b, pni, pei: (b, 0, 0)),
            pl.BlockSpec((1, M, D), lambda b, pni, pei: (b, 0, 0)),
            pl.BlockSpec((1, N, D), lambda b, pni, pei: (b, 0, 0)),
        ],
        scratch_shapes=[
            pltpu.VMEM((M, B, _K * 128), f32),
            pltpu.SemaphoreType.DMA,
        ],
    )

    out_shape = [
        jax.ShapeDtypeStruct((B, N, D), f32),
        jax.ShapeDtypeStruct((B, M, D), f32),
        jax.ShapeDtypeStruct((B, M, D), f32),
        jax.ShapeDtypeStruct((B, N, D), f32),
    ]

    Hn, ECn, ELn, Hm = pl.pallas_call(
        _egat_kernel,
        grid_spec=grid_spec,
        out_shape=out_shape,
        compiler_params=pltpu.CompilerParams(
            dimension_semantics=("arbitrary",),
        ),
    )(path_node_indices, path_edge_indices,
      H, ECtv, ELtv, AH, AEtv, MEtv, maskMM,
      W_H_w, W_H_b, W_EC_w, W_EC_b, W_EL_w, W_EL_b, aCr, aLr, bCr, bLr)

    return (Hn, ECn, ELn, Hm)


# speedup vs baseline: 3.1131x; 1.0278x over previous
import jax
import jax.numpy as jnp
import numpy as np
from jax.experimental import pallas as pl
from jax.experimental.pallas import tpu as pltpu

_NEG = -1e30
_K = 8


def _lrelu(x):
    return jnp.where(x >= 0, x, 0.2 * x)


def _dotT(a, b):
    return jax.lax.dot_general(a, b, (((0,), (0,)), ((), ())),
                               preferred_element_type=jnp.float32)


def _dotNT(a, b):
    return jax.lax.dot_general(a, b, (((1,), (1,)), ((), ())),
                               preferred_element_type=jnp.float32)


def _softmax_rows(score, adj_bool):
    masked = jnp.where(adj_bool, score, _NEG)
    m = jnp.max(masked, axis=1, keepdims=True)
    e = jnp.exp(masked - m)
    return e / jnp.sum(e, axis=1, keepdims=True)


def _egat_kernel(pni_ref, pei_ref,
                 H_ref, ECt_ref, ELt_ref, AH_ref, AEt_ref, MEt_ref, maskMM_ref,
                 WH_ref, WHb_ref, WEC_ref, WECb_ref, WEL_ref, WELb_ref,
                 aCr_ref, aLr_ref, bCr_ref, bLr_ref,
                 Hn_ref, ECn_ref, ELn_ref, Hm_ref,
                 rme_sc, dma_sem):
    N = 64
    M = 50
    D = 256
    B_ = 8
    Fe_ = 128
    b = pl.program_id(0)
    f32 = jnp.float32

    def _copies():
        for k in range(_K):
            icol = pni_ref[k] // 2
            yield pltpu.make_async_copy(
                MEt_ref.at[:, :, pl.ds(icol * 128, 128)],
                rme_sc.at[:, :, pl.ds(k * 128, 128)],
                dma_sem,
            )

    @pl.when(b == 0)
    def _():
        for c in _copies():
            c.start()

    H = H_ref[0]

    onehotB = (jax.lax.broadcasted_iota(jnp.int32, (B_, 1), 0) == b).astype(f32)
    selB = maskMM_ref[...] * jnp.broadcast_to(
        onehotB[None, :, :], (M, B_, 1)).reshape(M * B_, 1)
    ECall = ECt_ref[...].reshape(M * B_, Fe_)
    ELall = ELt_ref[...].reshape(M * B_, Fe_)
    EC = _dotT(selB, ECall)
    EL = _dotT(selB, ELall)
    aCr = aCr_ref[...]
    aLr = aLr_ref[...]
    bCr = bCr_ref[...]
    bLr = bLr_ref[...]
    WECb = WECb_ref[...].reshape(1, D)
    WELb = WELb_ref[...].reshape(1, D)

    Ht = jnp.dot(H, WH_ref[...], preferred_element_type=f32) + WHb_ref[...].reshape(1, D)
    FC = jnp.dot(EC, WEC_ref[...], preferred_element_type=f32)
    FL = jnp.dot(EL, WEL_ref[...], preferred_element_type=f32)

    wvC = _dotNT(WEC_ref[...], aCr[:, 2 * D:])
    wvL = _dotNT(WEL_ref[...], aLr[:, 2 * D:])
    fc = jnp.dot(EC, wvC, preferred_element_type=f32)
    fl = jnp.dot(EL, wvL, preferred_element_type=f32)
    fcfl = jnp.concatenate([fc, fl], axis=1)
    v_rowC = _dotNT(aCr[:, D:2 * D], Ht)
    v_rowL = _dotNT(aLr[:, D:2 * D], Ht)
    cstC = _dotNT(WECb, aCr[:, 2 * D:])
    cstL = _dotNT(WELb, aLr[:, 2 * D:])

    iota = jax.lax.broadcasted_iota(jnp.int32, (N, 1), 0)
    i_row = jnp.concatenate(
        [jnp.full((1, 1), pni_ref[k], jnp.int32) for k in range(_K)], axis=1)
    onehot = (iota == i_row).astype(f32)

    AHf = (AH_ref[0] > 0).astype(f32)
    adj8 = _dotT(onehot, AHf)
    any_adj = jnp.max(AHf, axis=1, keepdims=True)
    any8 = _dotT(onehot, any_adj)
    rowsH8 = _dotT(onehot, Ht)
    u8C = _dotNT(rowsH8, aCr[:, :D])
    u8L = _dotNT(rowsH8, aLr[:, :D])

    ECt = FC + WECb
    ELt = FL + WELb
    y_rowC = _dotNT(bCr[:, D:2 * D], ECt)
    y_rowL = _dotNT(bLr[:, D:2 * D], ELt)
    zC = _dotNT(H[0:1, :], bCr[:, 2 * D:])
    zL = _dotNT(H[0:1, :], bLr[:, 2 * D:])

    iotaE = jax.lax.broadcasted_iota(jnp.int32, (M, 1), 0)
    p_row = jnp.concatenate(
        [jnp.full((1, 1), pei_ref[k], jnp.int32) for k in range(_K)], axis=1)
    onehotE = (iotaE == p_row).astype(f32)
    AEf = (AEt_ref[:, 0, :] > 0).astype(f32)
    adjE8 = _dotT(onehotE, AEf)
    anyE = jnp.max(AEf, axis=1, keepdims=True)
    anyE8 = _dotT(onehotE, anyE)
    rowsEC8 = _dotT(onehotE, ECt)
    rowsEL8 = _dotT(onehotE, ELt)
    x8C = _dotNT(rowsEC8, bCr[:, :D])
    x8L = _dotNT(rowsEL8, bLr[:, :D])

    sc8EC = _lrelu(x8C + y_rowC + zC)
    sc8EL = _lrelu(x8L + y_rowL + zL)
    attn8EC = _softmax_rows(sc8EC, adjE8 > 0.5)
    attn8EL = _softmax_rows(sc8EL, adjE8 > 0.5)
    aggEC8 = jnp.dot(attn8EC, ECt, preferred_element_type=f32)
    aggEL8 = jnp.dot(attn8EL, ELt, preferred_element_type=f32)
    valEC = jnp.where(anyE8 > 0.5, aggEC8, rowsEC8)
    valEL = jnp.where(anyE8 > 0.5, aggEL8, rowsEL8)

    ECn_ref[0] = ECt
    ELn_ref[0] = ELt
    for k in range(_K):
        p = pei_ref[k]
        ECn_ref[0, pl.ds(p, 1), :] = valEC[k:k + 1, :]
        ELn_ref[0, pl.ds(p, 1), :] = valEL[k:k + 1, :]

    @pl.when(b == 0)
    def _():
        for c in _copies():
            c.wait()

    odd = [jax.lax.rem(pni_ref[k], 2) == 1 for k in range(_K)]
    rme_all = rme_sc[...].reshape(M * B_, _K * 128)
    blkball = _dotT(selB, rme_all)
    blks = [blkball[:, k * 128:(k + 1) * 128] for k in range(_K)]
    w_rows_C = []
    w_rows_L = []
    for k in range(_K):
        wk = _dotT(fcfl, blks[k])
        w_rows_C.append(jnp.where(odd[k], wk[0:1, N:], wk[0:1, :N]))
        w_rows_L.append(jnp.where(odd[k], wk[1:2, N:], wk[1:2, :N]))
    w8C = jnp.concatenate(w_rows_C, axis=0)
    w8L = jnp.concatenate(w_rows_L, axis=0)

    sc8C = _lrelu(u8C + v_rowC + w8C + cstC)
    sc8L = _lrelu(u8L + v_rowL + w8L + cstL)
    adjb = adj8 > 0.5
    attn8C = _softmax_rows(sc8C, adjb)
    attn8L = _softmax_rows(sc8L, adjb)
    aggC8 = jnp.dot(attn8C, Ht, preferred_element_type=f32)
    aggL8 = jnp.dot(attn8L, Ht, preferred_element_type=f32)
    valH = jnp.where(any8 > 0.5, 0.5 * (aggC8 + aggL8), rowsH8)

    ECr2all = _dotT(blkball, FC)
    ELr2all = _dotT(blkball, FL)
    ECr_sel = jnp.concatenate(
        [jnp.where(odd[k], ECr2all[k * 128 + N:(k + 1) * 128, :],
                   ECr2all[k * 128:k * 128 + N, :]) for k in range(_K)], axis=0)
    ELr_sel = jnp.concatenate(
        [jnp.where(odd[k], ELr2all[k * 128 + N:(k + 1) * 128, :],
                   ELr2all[k * 128:k * 128 + N, :]) for k in range(_K)], axis=0)
    Ht_tiled = jnp.broadcast_to(Ht[None, :, :], (_K, N, D)).reshape(_K * N, D)
    XC = Ht_tiled * ECr_sel
    XL = Ht_tiled * ELr_sel
    bd = (jax.lax.broadcasted_iota(jnp.int32, (_K, _K * N), 1) // N
          == jax.lax.broadcasted_iota(jnp.int32, (_K, _K * N), 0)).astype(f32)
    PC = jnp.tile(attn8C, (1, _K)) * bd
    PL = jnp.tile(attn8L, (1, _K)) * bd
    mkC8 = jnp.dot(PC, XC, preferred_element_type=f32) + WECb * aggC8
    mkL8 = jnp.dot(PL, XL, preferred_element_type=f32) + WELb * aggL8
    mk8 = 0.5 * (mkC8 + mkL8) * any8

    Hn_ref[0] = Ht
    Hm_ref[0] = jnp.zeros((N, D), dtype=f32)
    for k in range(_K):
        i = pni_ref[k]
        Hm_ref[0, pl.ds(i, 1), :] = mk8[k:k + 1, :]
        Hn_ref[0, pl.ds(i, 1), :] = valH[k:k + 1, :]


def kernel(H, E_C, E_L, AH, AE, ME, MH, path_node_indices, path_edge_indices,
           W_H_w, W_H_b, W_EC_w, W_EC_b, W_EL_w, W_EL_b, a_C, a_L, b_C, b_L):
    B, N, ND = H.shape
    M = E_C.shape[1]
    Fe = E_C.shape[2]
    D = W_H_w.shape[1]
    A3 = a_C.shape[0]
    f32 = jnp.float32

    ECtv = jnp.transpose(E_C, (1, 0, 2))
    ELtv = jnp.transpose(E_L, (1, 0, 2))
    AEtv = jnp.transpose(AE, (1, 0, 2))
    MEtv = jnp.transpose(ME, (2, 0, 1))
    maskMM = jnp.asarray((np.arange(M * B)[:, None] // B
                          == np.arange(M)[None, :]).astype(np.float32))
    aCr = jnp.transpose(a_C, (1, 0))
    aLr = jnp.transpose(a_L, (1, 0))
    bCr = jnp.transpose(b_C, (1, 0))
    bLr = jnp.transpose(b_L, (1, 0))

    grid_spec = pltpu.PrefetchScalarGridSpec(
        num_scalar_prefetch=2,
        grid=(B,),
        in_specs=[
            pl.BlockSpec((1, N, ND), lambda b, pni, pei: (b, 0, 0)),
            pl.BlockSpec((M, B, Fe), lambda b, pni, pei: (0, 0, 0)),
            pl.BlockSpec((M, B, Fe), lambda b, pni, pei: (0, 0, 0)),
            pl.BlockSpec((1, N, N), lambda b, pni, pei: (0, 0, 0)),
            pl.BlockSpec((M, B, M), lambda b, pni, pei: (0, 0, 0)),
            pl.BlockSpec(memory_space=pltpu.MemorySpace.HBM),
            pl.BlockSpec((M * B, M), lambda b, pni, pei: (0, 0)),
            pl.BlockSpec((ND, D), lambda b, pni, pei: (0, 0)),
            pl.BlockSpec((D,), lambda b, pni, pei: (0,)),
            pl.BlockSpec((Fe, D), lambda b, pni, pei: (0, 0)),
            pl.BlockSpec((D,), lambda b, pni, pei: (0,)),
            pl.BlockSpec((Fe, D), lambda b, pni, pei: (0, 0)),
            pl.BlockSpec((D,), lambda b, pni, pei: (0,)),
            pl.BlockSpec((1, A3), lambda b, pni, pei: (0, 0)),
            pl.BlockSpec((1, A3), lambda b, pni, pei: (0, 0)),
            pl.BlockSpec((1, A3), lambda b, pni, pei: (0, 0)),
            pl.BlockSpec((1, A3), lambda b, pni, pei: (0, 0)),
        ],
        out_specs=[
            pl.BlockSpec((1, N, D), lambda b, pni, pei: (b, 0, 0)),
            pl.BlockSpec((1, M, D), lambda b, pni, pei: (b, 0, 0)),
            pl.BlockSpec((1, M, D), lambda b, pni, pei: (b, 0, 0)),
            pl.BlockSpec((1, N, D), lambda b, pni, pei: (b, 0, 0)),
        ],
        scratch_shapes=[
            pltpu.VMEM((M, B, _K * 128), f32),
            pltpu.SemaphoreType.DMA,
        ],
    )

    out_shape = [
        jax.ShapeDtypeStruct((B, N, D), f32),
        jax.ShapeDtypeStruct((B, M, D), f32),
        jax.ShapeDtypeStruct((B, M, D), f32),
        jax.ShapeDtypeStruct((B, N, D), f32),
    ]

    Hn, ECn, ELn, Hm = pl.pallas_call(
        _egat_kernel,
        grid_spec=grid_spec,
        out_shape=out_shape,
        compiler_params=pltpu.CompilerParams(
            dimension_semantics=("arbitrary",),
        ),
    )(path_node_indices, path_edge_indices,
      H, ECtv, ELtv, AH, AEtv, MEtv, maskMM,
      W_H_w, W_H_b, W_EC_w, W_EC_b, W_EL_w, W_EL_b, aCr, aLr, bCr, bLr)

    return (Hn, ECn, ELn, Hm)
